# Initial kernel scaffold; baseline (speedup 1.0000x reference)
import jax
import jax.numpy as jnp
from jax.experimental import pallas as pl


def kernel(x, table, weights, hash0_coefs, hash1_coefs):
    raise NotImplementedError("write your pallas kernel here")


# speedup vs baseline: 5.0700x; 5.0700x over previous
import functools

import jax
import jax.numpy as jnp
from jax import lax
from jax.experimental import pallas as pl
from jax.experimental.pallas import tpu as pltpu
from jax.experimental.pallas import tpu_sc as plsc

PRIME = (1 << 31) - 1
BTAB = 1 << 20
KW = 149796
DIM = 64
DSUB = 56
NH = 8
BATCH = 16384
NC, NS, L = 2, 16, 16
NW = NC * NS
EPW = BATCH // NW
CH = 16
NCHUNK = EPW // CH
ROWS = CH * NH
WG = 4
_OFFS = (0, 16, 32, 40)


def _red(v):
    return (v & jnp.uint32(PRIME)) + (v >> jnp.uint32(31))


def _red2(v):
    return _red(_red(v))


def _mulshift(t, s):
    th = t >> jnp.uint32(31 - s)
    tl = t & jnp.uint32((1 << (31 - s)) - 1)
    return th + (tl << jnp.uint32(s))


def _hash_mod_p(a_u, b0_u, b1_u, x0, x1):
    t1 = _red(_mulshift(b1_u * x1, 26))
    t2 = _red(_mulshift(b1_u * x0, 16))
    t3 = _red(_mulshift(b0_u * x1, 10))
    t4 = _red(b0_u * x0)
    s = _red2(_red2(t1 + t2) + _red2(t3 + t4))
    h = _red2(s + a_u)
    return jnp.where(h == jnp.uint32(PRIME), jnp.uint32(0), h)


def _sc_body(x_hbm, coef_hbm, table_hbm, w_hbm, out_hbm,
             x_v, coef_v, idx0_v, idx1_v, par_v, w_v, rows_v, out_v, tmp_v,
             sem, wsem):
    wid = lax.axis_index("s") * NC + lax.axis_index("c")
    base = wid * EPW

    pltpu.sync_copy(x_hbm.at[pl.ds(base, EPW)], x_v)
    pltpu.sync_copy(coef_hbm, coef_v)

    cv0 = coef_v[pl.ds(0, L)]
    cv1 = coef_v[pl.ds(L, L)]
    abs_ = []
    for j in range(NH + 1):
        a = (cv0[2 * j] if 2 * j < L else cv1[2 * j - L]).astype(jnp.uint32)
        b = (cv0[2 * j + 1] if 2 * j + 1 < L else cv1[2 * j + 1 - L]).astype(jnp.uint32)
        abs_.append((a, b & jnp.uint32(0xFFFF), b >> jnp.uint32(16)))

    inv_k = jnp.float32(1.0) / jnp.float32(KW)

    def hash_body(c, carry):
        xv = x_v[pl.ds(c * jnp.int32(L), L)].astype(jnp.uint32)
        x0 = xv & jnp.uint32(0x3FF)
        x1 = xv >> jnp.uint32(10)
        for j in range(NH):
            a_u, b0_u, b1_u = abs_[j]
            h = _hash_mod_p(a_u, b0_u, b1_u, x0, x1)
            idx0_v[c, pl.ds(j * L, L)] = (h & jnp.uint32(BTAB - 1)).astype(jnp.int32)
        a_u, b0_u, b1_u = abs_[NH]
        h1 = _hash_mod_p(a_u, b0_u, b1_u, x0, x1)
        q = (h1.astype(jnp.float32) * inv_k).astype(jnp.int32)
        r = h1.astype(jnp.int32) - q * jnp.int32(KW)
        kw = jnp.int32(KW)
        z = jnp.int32(0)
        r = jnp.where(r < z, r + kw, r)
        r = jnp.where(r >= kw, r - kw, r)
        r = jnp.where(r < z, r + kw, r)
        r = jnp.where(r >= kw, r - kw, r)
        idx1_v[lax.shift_right_logical(c, jnp.int32(3)),
               pl.ds((c & jnp.int32(7)) * jnp.int32(L), L)] = (
                   lax.shift_right_logical(r, jnp.int32(1)))
        par_v[pl.ds(c * jnp.int32(L), L)] = r & jnp.int32(1)
        return carry

    lax.fori_loop(jnp.int32(0), jnp.int32(NCHUNK), hash_body, jnp.int32(0))

    for g in range(WG):
        pltpu.async_copy(
            w_hbm.at[idx1_v.at[jnp.int32(g)]],
            w_v.at[pl.ds(jnp.int32(g * (EPW // WG)), EPW // WG)], wsem).wait()

    def chunk_body(c, carry):
        pltpu.async_copy(table_hbm.at[idx0_v.at[c]], rows_v, sem).wait()
        parv = par_v[pl.ds(c * jnp.int32(L), L)]
        for e in range(CH):
            n = c * jnp.int32(CH) + jnp.int32(e)
            nd = n * jnp.int32(DIM)
            wv = w_v[n, pl.ds(0, L)]
            p = parv[e]
            odd = p != jnp.int32(0)
            ws = [jnp.where(odd, wv[NH + j], wv[j]) for j in range(NH)]
            accs = []
            for off in _OFFS:
                acc = rows_v[e, pl.ds(off, L)] * ws[0]
                for j in range(1, NH):
                    acc = acc + rows_v[j * CH + e, pl.ds(off, L)] * ws[j]
                accs.append(acc)
            out_v[pl.ds(nd, L)] = accs[0]
            out_v[pl.ds(nd + jnp.int32(16), L)] = accs[1]
            out_v[pl.ds(nd + jnp.int32(32), L)] = accs[2]
            tmp_v[pl.ds(jnp.int32(16) - p * jnp.int32(NH), L)] = wv
            tmp_v[pl.ds(0, L)] = accs[3]
            out_v[pl.ds(nd + jnp.int32(48), L)] = tmp_v[pl.ds(8, L)]
        return carry

    lax.fori_loop(jnp.int32(0), jnp.int32(NCHUNK), chunk_body, jnp.int32(0))

    pltpu.sync_copy(out_v, out_hbm.at[pl.ds(base * jnp.int32(DIM), EPW * DIM)])


@jax.jit
def kernel(x, table, weights, hash0_coefs, hash1_coefs):
    x32 = x.astype(jnp.int32)
    coefs = jnp.concatenate(
        [hash0_coefs.reshape(-1), hash1_coefs.reshape(-1),
         jnp.zeros((14,), hash0_coefs.dtype)]).astype(jnp.int32)

    mesh = plsc.VectorSubcoreMesh(
        core_axis_name="c", subcore_axis_name="s",
        num_cores=NC, num_subcores=NS)

    fn = pl.kernel(
        _sc_body,
        out_type=jax.ShapeDtypeStruct((BATCH * DIM,), jnp.float32),
        mesh=mesh,
        compiler_params=pltpu.CompilerParams(use_tc_tiling_on_sc=False),
        scratch_types=[
            pltpu.VMEM((EPW,), jnp.int32),
            pltpu.VMEM((2 * (NH + 1) + 14,), jnp.int32),
            pltpu.VMEM((NCHUNK, ROWS), jnp.int32),
            pltpu.VMEM((WG, EPW // WG), jnp.int32),
            pltpu.VMEM((EPW,), jnp.int32),
            pltpu.VMEM((EPW, L), jnp.float32),
            pltpu.VMEM((ROWS, DSUB), jnp.float32),
            pltpu.VMEM((EPW * DIM,), jnp.float32),
            pltpu.VMEM((2 * L,), jnp.float32),
            pltpu.SemaphoreType.DMA,
            pltpu.SemaphoreType.DMA,
        ],
    )
    out = fn(x32, coefs, table, weights.reshape(KW // 2, 2 * NH))
    return out.reshape(BATCH, DIM)


# speedup vs baseline: 5.0752x; 1.0010x over previous
import functools

import jax
import jax.numpy as jnp
from jax import lax
from jax.experimental import pallas as pl
from jax.experimental.pallas import tpu as pltpu
from jax.experimental.pallas import tpu_sc as plsc

PRIME = (1 << 31) - 1
BTAB = 1 << 20
KW = 149796
DIM = 64
DSUB = 56
NH = 8
BATCH = 16384
NC, NS, L = 2, 16, 16
NW = NC * NS
EPW = BATCH // NW
CH = 16
NCHUNK = EPW // CH
ROWS = CH * NH
WG = 4
NBUF = 4
NSPL = 4
RSPL = ROWS // NSPL
_OFFS = (0, 16, 32, 40)


def _red(v):
    return (v & jnp.uint32(PRIME)) + (v >> jnp.uint32(31))


def _red2(v):
    return _red(_red(v))


def _mulshift(t, s):
    th = t >> jnp.uint32(31 - s)
    tl = t & jnp.uint32((1 << (31 - s)) - 1)
    return th + (tl << jnp.uint32(s))


def _hash_mod_p(a_u, b0_u, b1_u, x0, x1):
    t1 = _red(_mulshift(b1_u * x1, 26))
    t2 = _red(_mulshift(b1_u * x0, 16))
    t3 = _red(_mulshift(b0_u * x1, 10))
    t4 = _red(b0_u * x0)
    s = _red2(_red2(t1 + t2) + _red2(t3 + t4))
    h = _red2(s + a_u)
    return jnp.where(h == jnp.uint32(PRIME), jnp.uint32(0), h)


def _sc_body(x_hbm, coef_hbm, table_hbm, w_hbm, out_hbm,
             x_v, coef_v, idx0_v, idx1_v, par_v, w_v, rows_v0, rows_v1,
             rows_v2, rows_v3, out_v, tmp_v, sem0, sem1, sem2, sem3, wsem):
    wid = lax.axis_index("s") * NC + lax.axis_index("c")
    base = wid * EPW

    pltpu.sync_copy(x_hbm.at[pl.ds(base, EPW)], x_v)
    pltpu.sync_copy(coef_hbm, coef_v)

    cv0 = coef_v[pl.ds(0, L)]
    cv1 = coef_v[pl.ds(L, L)]
    abs_ = []
    for j in range(NH + 1):
        a = (cv0[2 * j] if 2 * j < L else cv1[2 * j - L]).astype(jnp.uint32)
        b = (cv0[2 * j + 1] if 2 * j + 1 < L else cv1[2 * j + 1 - L]).astype(jnp.uint32)
        abs_.append((a, b & jnp.uint32(0xFFFF), b >> jnp.uint32(16)))

    inv_k = jnp.float32(1.0) / jnp.float32(KW)

    def hash_body(c, carry):
        xv = x_v[pl.ds(c * jnp.int32(L), L)].astype(jnp.uint32)
        x0 = xv & jnp.uint32(0x3FF)
        x1 = xv >> jnp.uint32(10)
        for j in range(NH):
            a_u, b0_u, b1_u = abs_[j]
            h = _hash_mod_p(a_u, b0_u, b1_u, x0, x1)
            idx0_v[c, pl.ds(j * L, L)] = (h & jnp.uint32(BTAB - 1)).astype(jnp.int32)
        a_u, b0_u, b1_u = abs_[NH]
        h1 = _hash_mod_p(a_u, b0_u, b1_u, x0, x1)
        q = (h1.astype(jnp.float32) * inv_k).astype(jnp.int32)
        r = h1.astype(jnp.int32) - q * jnp.int32(KW)
        kw = jnp.int32(KW)
        z = jnp.int32(0)
        r = jnp.where(r < z, r + kw, r)
        r = jnp.where(r >= kw, r - kw, r)
        r = jnp.where(r < z, r + kw, r)
        r = jnp.where(r >= kw, r - kw, r)
        idx1_v[lax.shift_right_logical(c, jnp.int32(3)),
               pl.ds((c & jnp.int32(7)) * jnp.int32(L), L)] = (
                   lax.shift_right_logical(r, jnp.int32(1)))
        par_v[pl.ds(c * jnp.int32(L), L)] = r & jnp.int32(1)
        return carry

    lax.fori_loop(jnp.int32(0), jnp.int32(NCHUNK), hash_body, jnp.int32(0))

    for g in range(WG):
        pltpu.async_copy(
            w_hbm.at[idx1_v.at[jnp.int32(g)]],
            w_v.at[pl.ds(jnp.int32(g * (EPW // WG)), EPW // WG)], wsem)
    rows_bufs = (rows_v0, rows_v1, rows_v2, rows_v3)
    sems = (sem0, sem1, sem2, sem3)
    def start_chunk(c, b):
        for s in range(NSPL):
            pltpu.async_copy(
                table_hbm.at[idx0_v.at[c, pl.ds(jnp.int32(s * RSPL), RSPL)]],
                rows_bufs[b].at[pl.ds(jnp.int32(s * RSPL), RSPL)], sems[b])

    def wait_chunk(c, b):
        for s in range(NSPL):
            pltpu.make_async_copy(
                table_hbm.at[idx0_v.at[c, pl.ds(jnp.int32(s * RSPL), RSPL)]],
                rows_bufs[b].at[pl.ds(jnp.int32(s * RSPL), RSPL)],
                sems[b]).wait()

    for b in range(NBUF):
        start_chunk(jnp.int32(b), b)
    for g in range(WG):
        pltpu.make_async_copy(
            w_hbm.at[idx1_v.at[jnp.int32(g)]],
            w_v.at[pl.ds(jnp.int32(g * (EPW // WG)), EPW // WG)], wsem).wait()

    def compute_chunk(c, rows_v):
        parv = par_v[pl.ds(c * jnp.int32(L), L)]
        for e in range(CH):
            n = c * jnp.int32(CH) + jnp.int32(e)
            nd = n * jnp.int32(DIM)
            wv = w_v[n, pl.ds(0, L)]
            p = parv[e]
            odd = p != jnp.int32(0)
            ws = [jnp.where(odd, wv[NH + j], wv[j]) for j in range(NH)]
            accs = []
            for off in _OFFS:
                acc = rows_v[e, pl.ds(off, L)] * ws[0]
                for j in range(1, NH):
                    acc = acc + rows_v[j * CH + e, pl.ds(off, L)] * ws[j]
                accs.append(acc)
            out_v[pl.ds(nd, L)] = accs[0]
            out_v[pl.ds(nd + jnp.int32(16), L)] = accs[1]
            out_v[pl.ds(nd + jnp.int32(32), L)] = accs[2]
            tmp_v[pl.ds(jnp.int32(16) - p * jnp.int32(NH), L)] = wv
            tmp_v[pl.ds(0, L)] = accs[3]
            out_v[pl.ds(nd + jnp.int32(48), L)] = tmp_v[pl.ds(8, L)]

    def ring_body(i, carry):
        for b in range(NBUF):
            c = i * jnp.int32(NBUF) + jnp.int32(b)
            wait_chunk(c, b)
            compute_chunk(c, rows_bufs[b])
            nxt = c + jnp.int32(NBUF)

            @pl.when(nxt < jnp.int32(NCHUNK))
            def _():
                start_chunk(nxt, b)
        return carry

    lax.fori_loop(jnp.int32(0), jnp.int32(NCHUNK // NBUF), ring_body,
                  jnp.int32(0))

    pltpu.sync_copy(out_v, out_hbm.at[pl.ds(base * jnp.int32(DIM), EPW * DIM)])


@jax.jit
def kernel(x, table, weights, hash0_coefs, hash1_coefs):
    x32 = x.astype(jnp.int32)
    coefs = jnp.concatenate(
        [hash0_coefs.reshape(-1), hash1_coefs.reshape(-1),
         jnp.zeros((14,), hash0_coefs.dtype)]).astype(jnp.int32)

    mesh = plsc.VectorSubcoreMesh(
        core_axis_name="c", subcore_axis_name="s",
        num_cores=NC, num_subcores=NS)

    fn = pl.kernel(
        _sc_body,
        out_type=jax.ShapeDtypeStruct((BATCH * DIM,), jnp.float32),
        mesh=mesh,
        compiler_params=pltpu.CompilerParams(use_tc_tiling_on_sc=False),
        scratch_types=[
            pltpu.VMEM((EPW,), jnp.int32),
            pltpu.VMEM((2 * (NH + 1) + 14,), jnp.int32),
            pltpu.VMEM((NCHUNK, ROWS), jnp.int32),
            pltpu.VMEM((WG, EPW // WG), jnp.int32),
            pltpu.VMEM((EPW,), jnp.int32),
            pltpu.VMEM((EPW, L), jnp.float32),
            pltpu.VMEM((ROWS, DSUB), jnp.float32),
            pltpu.VMEM((ROWS, DSUB), jnp.float32),
            pltpu.VMEM((ROWS, DSUB), jnp.float32),
            pltpu.VMEM((ROWS, DSUB), jnp.float32),
            pltpu.VMEM((EPW * DIM,), jnp.float32),
            pltpu.VMEM((2 * L,), jnp.float32),
            pltpu.SemaphoreType.DMA,
            pltpu.SemaphoreType.DMA,
            pltpu.SemaphoreType.DMA,
            pltpu.SemaphoreType.DMA,
            pltpu.SemaphoreType.DMA,
        ],
    )
    out = fn(x32, coefs, table, weights.reshape(KW // 2, 2 * NH))
    return out.reshape(BATCH, DIM)


# speedup vs baseline: 5.0764x; 1.0002x over previous
import functools

import jax
import jax.numpy as jnp
from jax import lax
from jax.experimental import pallas as pl
from jax.experimental.pallas import tpu as pltpu
from jax.experimental.pallas import tpu_sc as plsc

PRIME = (1 << 31) - 1
BTAB = 1 << 20
KW = 149796
DIM = 64
DSUB = 56
NH = 8
BATCH = 16384
NC, NS, L = 2, 16, 16
NW = NC * NS
EPW = BATCH // NW
CH = 16
NCHUNK = EPW // CH
ROWS = CH * NH
WG = 4
NBUF = 4
NSPL = 4
RSPL = ROWS // NSPL
_OFFS = (0, 16, 32, 40)


def _red(v):
    return (v & jnp.uint32(PRIME)) + (v >> jnp.uint32(31))


def _red2(v):
    return _red(_red(v))


def _mulshift(t, s):
    th = t >> jnp.uint32(31 - s)
    tl = t & jnp.uint32((1 << (31 - s)) - 1)
    return th + (tl << jnp.uint32(s))


def _hash_mod_p(a_u, b0_u, b1_u, x0, x1):
    t1 = _red(_mulshift(b1_u * x1, 26))
    t2 = _red(_mulshift(b1_u * x0, 16))
    t3 = _red(_mulshift(b0_u * x1, 10))
    t4 = _red(b0_u * x0)
    s = _red2(_red2(t1 + t2) + _red2(t3 + t4))
    h = _red2(s + a_u)
    return jnp.where(h == jnp.uint32(PRIME), jnp.uint32(0), h)


def _sc_body(x_hbm, coef_hbm, table_hbm, w_hbm, out_hbm,
             x_v, coef_v, idx0_v, idx1_v, par_v, w_v, rows_v0, rows_v1,
             rows_v2, rows_v3, out_v, tmp_v, sem0, sem1, sem2, sem3, wsem):
    wid = lax.axis_index("s") * NC + lax.axis_index("c")
    base = wid * EPW

    pltpu.sync_copy(x_hbm.at[pl.ds(base, EPW)], x_v)
    pltpu.sync_copy(coef_hbm, coef_v)

    cv0 = coef_v[pl.ds(0, L)]
    cv1 = coef_v[pl.ds(L, L)]
    abs_ = []
    for j in range(NH + 1):
        a = (cv0[2 * j] if 2 * j < L else cv1[2 * j - L]).astype(jnp.uint32)
        b = (cv0[2 * j + 1] if 2 * j + 1 < L else cv1[2 * j + 1 - L]).astype(jnp.uint32)
        abs_.append((a, b & jnp.uint32(0xFFFF), b >> jnp.uint32(16)))

    inv_k = jnp.float32(1.0) / jnp.float32(KW)

    def hash_body(c, carry):
        xv = x_v[pl.ds(c * jnp.int32(L), L)].astype(jnp.uint32)
        x0 = xv & jnp.uint32(0x3FF)
        x1 = xv >> jnp.uint32(10)
        for j in range(NH):
            a_u, b0_u, b1_u = abs_[j]
            h = _hash_mod_p(a_u, b0_u, b1_u, x0, x1)
            idx0_v[c, pl.ds(j * L, L)] = (h & jnp.uint32(BTAB - 1)).astype(jnp.int32)
        a_u, b0_u, b1_u = abs_[NH]
        h1 = _hash_mod_p(a_u, b0_u, b1_u, x0, x1)
        q = (h1.astype(jnp.float32) * inv_k).astype(jnp.int32)
        r = h1.astype(jnp.int32) - q * jnp.int32(KW)
        kw = jnp.int32(KW)
        z = jnp.int32(0)
        r = jnp.where(r < z, r + kw, r)
        r = jnp.where(r >= kw, r - kw, r)
        r = jnp.where(r < z, r + kw, r)
        r = jnp.where(r >= kw, r - kw, r)
        idx1_v[lax.shift_right_logical(c, jnp.int32(3)),
               pl.ds((c & jnp.int32(7)) * jnp.int32(L), L)] = (
                   lax.shift_right_logical(r, jnp.int32(1)))
        par_v[pl.ds(c * jnp.int32(L), L)] = r & jnp.int32(1)
        return carry

    lax.fori_loop(jnp.int32(0), jnp.int32(NCHUNK), hash_body, jnp.int32(0))

    for g in range(WG):
        pltpu.async_copy(
            w_hbm.at[idx1_v.at[jnp.int32(g)]],
            w_v.at[pl.ds(jnp.int32(g * (EPW // WG)), EPW // WG)], wsem)
    rows_bufs = (rows_v0, rows_v1, rows_v2, rows_v3)
    sems = (sem0, sem1, sem2, sem3)
    def start_chunk(c, b):
        for s in range(NSPL):
            pltpu.async_copy(
                table_hbm.at[idx0_v.at[c, pl.ds(jnp.int32(s * RSPL), RSPL)]],
                rows_bufs[b].at[pl.ds(jnp.int32(s * RSPL), RSPL)], sems[b])

    def wait_chunk(c, b):
        for s in range(NSPL):
            pltpu.make_async_copy(
                table_hbm.at[idx0_v.at[c, pl.ds(jnp.int32(s * RSPL), RSPL)]],
                rows_bufs[b].at[pl.ds(jnp.int32(s * RSPL), RSPL)],
                sems[b]).wait()

    for b in range(NBUF):
        start_chunk(jnp.int32(b), b)
    for g in range(WG):
        pltpu.make_async_copy(
            w_hbm.at[idx1_v.at[jnp.int32(g)]],
            w_v.at[pl.ds(jnp.int32(g * (EPW // WG)), EPW // WG)], wsem).wait()

    def compute_chunk(c, rows_v):
        parv = par_v[pl.ds(c * jnp.int32(L), L)]
        for e in range(CH):
            n = c * jnp.int32(CH) + jnp.int32(e)
            nd = n * jnp.int32(DIM)
            wv = w_v[n, pl.ds(0, L)]
            p = parv[e]
            odd = p != jnp.int32(0)
            ws = [jnp.where(odd, wv[NH + j], wv[j]) for j in range(NH)]
            accs = []
            for off in _OFFS:
                acc = rows_v[e, pl.ds(off, L)] * ws[0]
                for j in range(1, NH):
                    acc = acc + rows_v[j * CH + e, pl.ds(off, L)] * ws[j]
                accs.append(acc)
            out_v[pl.ds(nd, L)] = accs[0]
            out_v[pl.ds(nd + jnp.int32(16), L)] = accs[1]
            out_v[pl.ds(nd + jnp.int32(32), L)] = accs[2]
            tmp_v[pl.ds(jnp.int32(16) - p * jnp.int32(NH), L)] = wv
            tmp_v[pl.ds(0, L)] = accs[3]
            out_v[pl.ds(nd + jnp.int32(48), L)] = tmp_v[pl.ds(8, L)]

    def ring_body(i, carry):
        for b in range(NBUF):
            c = i * jnp.int32(NBUF) + jnp.int32(b)
            wait_chunk(c, b)
            compute_chunk(c, rows_bufs[b])
            nxt = c + jnp.int32(NBUF)

            @pl.when(nxt < jnp.int32(NCHUNK))
            def _():
                start_chunk(nxt, b)
        return carry

    lax.fori_loop(jnp.int32(0), jnp.int32(NCHUNK // NBUF), ring_body,
                  jnp.int32(0))

    pltpu.sync_copy(out_v, out_hbm.at[pl.ds(base * jnp.int32(DIM), EPW * DIM)])


@jax.jit
def kernel(x, table, weights, hash0_coefs, hash1_coefs):
    x32 = x.astype(jnp.int32)
    coefs = jnp.concatenate(
        [hash0_coefs.reshape(-1), hash1_coefs.reshape(-1),
         jnp.zeros((14,), hash0_coefs.dtype)]).astype(jnp.int32)

    mesh = plsc.VectorSubcoreMesh(
        core_axis_name="c", subcore_axis_name="s",
        num_cores=NC, num_subcores=NS)

    fn = pl.kernel(
        _sc_body,
        out_type=jax.ShapeDtypeStruct((BATCH * DIM,), jnp.float32),
        mesh=mesh,
        compiler_params=pltpu.CompilerParams(use_tc_tiling_on_sc=False),
        scratch_types=[
            pltpu.VMEM((EPW,), jnp.int32),
            pltpu.VMEM((2 * (NH + 1) + 14,), jnp.int32),
            pltpu.VMEM((NCHUNK, ROWS), jnp.int32),
            pltpu.VMEM((WG, EPW // WG), jnp.int32),
            pltpu.VMEM((EPW,), jnp.int32),
            pltpu.VMEM((EPW, L), jnp.float32),
            pltpu.VMEM((ROWS, DSUB), jnp.float32),
            pltpu.VMEM((ROWS, DSUB), jnp.float32),
            pltpu.VMEM((ROWS, DSUB), jnp.float32),
            pltpu.VMEM((ROWS, DSUB), jnp.float32),
            pltpu.VMEM((EPW * DIM,), jnp.float32),
            pltpu.VMEM((2 * L,), jnp.float32),
            pltpu.SemaphoreType.DMA,
            pltpu.SemaphoreType.DMA,
            pltpu.SemaphoreType.DMA,
            pltpu.SemaphoreType.DMA,
            pltpu.SemaphoreType.DMA,
        ],
    )
    table_lin = lax.optimization_barrier(
        table.reshape(-1)).reshape(BTAB, DSUB)
    out = fn(x32, coefs, table_lin, weights.reshape(KW // 2, 2 * NH))
    return out.reshape(BATCH, DIM)
